# baseline (speedup 1.0000x reference)
import functools

import jax
import jax.numpy as jnp
from jax import lax
from jax.experimental import pallas as pl
from jax.experimental.pallas import tpu as pltpu
from jax.experimental.pallas import tpu_sc as plsc

K = 8
TILE_A = 1024
TILE_C = 1024


def _knn_body(xc_ref, x_ref, wn_ref, wc_ref, b1_ref,
              idx_ref, y_ref, z_ref):
    n_total = x_ref.shape[2]
    xc = xc_ref[0]
    xb = x_ref[0]
    cdims = (((0,), (0,)), ((), ()))
    s = lax.dot_general(xc, xb, cdims,
                        preferred_element_type=jnp.float32)
    inner = -2.0 * s
    xx_all = jnp.sum(xb * xb, axis=0, keepdims=True)
    xx_row = jnp.transpose(
        jnp.sum(xc * xc, axis=0, keepdims=True))
    p = (0.0 - xx_row) - inner - xx_all

    colidx = lax.broadcasted_iota(
        jnp.int32, p.shape, 1).astype(jnp.float32)
    neg = jnp.float32(-jnp.inf)
    big = jnp.float32(n_total)
    picks = []
    for _ in range(K):
        m = jnp.max(p, axis=1, keepdims=True)
        cand = jnp.where(p >= m, colidx, big)
        am = jnp.min(cand, axis=1, keepdims=True)
        picks.append(am)
        p = jnp.where(cand == am, neg, p)
    idx_ref[0] = jnp.transpose(
        jnp.concatenate(picks, axis=1)).astype(jnp.int32)

    y_ref[0] = lax.dot_general(xc, wn_ref[...], cdims,
                               preferred_element_type=jnp.float32)
    z_ref[0] = (lax.dot_general(xc, wc_ref[...], cdims,
                                preferred_element_type=jnp.float32)
                + b1_ref[...])


def _mlp_body(g_ref, z_ref, x_ref, w2_ref, b2_ref, w3_ref, b3_ref,
              wa_ref, ba_ref, out_ref):
    k, t, c = g_ref.shape
    g = g_ref[...]
    z = z_ref[...]
    h1 = jnp.maximum(g + z[None, :, :], 0.0)
    h1f = h1.reshape(k * t, c)
    h2 = jnp.maximum(
        jnp.dot(h1f, w2_ref[...], preferred_element_type=jnp.float32)
        + b2_ref[...], 0.0)
    h3 = jnp.maximum(
        jnp.dot(h2, w3_ref[...], preferred_element_type=jnp.float32)
        + b3_ref[...], 0.0).reshape(k, t, c)
    m = h3[0]
    for i in range(1, k):
        m = jnp.maximum(m, h3[i])
    pre = x_ref[0] + jnp.transpose(m)
    out_ref[...] = jnp.maximum(
        jnp.dot(wa_ref[...], pre, preferred_element_type=jnp.float32)
        + ba_ref[...], 0.0)


def _make_sc_gather(n_rows, d, n_idx):
    info = plsc.get_sparse_core_info()
    nc, ns = info.num_cores, info.num_subcores
    nw = nc * ns
    per_w = n_idx // nw
    chunk = 256
    n_chunks = per_w // chunk
    mesh = plsc.VectorSubcoreMesh(core_axis_name="c", subcore_axis_name="s")

    @functools.partial(
        pl.kernel, mesh=mesh,
        out_type=jax.ShapeDtypeStruct((n_idx, d), jnp.float32),
        scratch_types=[
            pltpu.VMEM((per_w,), jnp.int32),
            pltpu.VMEM((chunk, d), jnp.float32),
            pltpu.VMEM((chunk, d), jnp.float32),
            pltpu.SemaphoreType.DMA,
            pltpu.SemaphoreType.DMA,
        ],
    )
    def gather_kernel(table_hbm, idx_hbm, out_hbm, idx_v, rows0, rows1, s0, s1):
        wid = lax.axis_index("s") * nc + lax.axis_index("c")
        base = wid * per_w
        pltpu.sync_copy(idx_hbm.at[pl.ds(base, per_w)], idx_v)
        bufs = (rows0, rows1)
        sems = (s0, s1)
        descs = [None] * n_chunks
        descs[0] = pltpu.async_copy(
            table_hbm.at[idx_v.at[pl.ds(0, chunk)]], bufs[0], sems[0])
        for i in range(n_chunks):
            if i + 1 < n_chunks:
                descs[i + 1] = pltpu.async_copy(
                    table_hbm.at[idx_v.at[pl.ds((i + 1) * chunk, chunk)]],
                    bufs[(i + 1) % 2], sems[(i + 1) % 2])
            descs[i].wait()
            pltpu.sync_copy(bufs[i % 2],
                            out_hbm.at[pl.ds(base + i * chunk, chunk)])

    return gather_kernel


def kernel(x, W_s1, b_s1, W_s2, b_s2, W_s3, b_s3, W_a, b_a):
    b, c, n = x.shape
    groups = 32
    cpg = W_s2.shape[1]
    width = W_s1.shape[0]
    c_out = W_s3.shape[0]

    wn_t = jnp.transpose(W_s1[:, :c])
    wc_t = jnp.transpose(W_s1[:, c:])
    b1r = b_s1.reshape(1, width)

    bd_mask = jnp.kron(jnp.eye(groups, dtype=jnp.float32),
                       jnp.ones((cpg, cpg), jnp.float32))
    w2_tiled = jnp.tile(
        W_s2.reshape(groups, cpg, cpg).transpose(2, 0, 1).reshape(
            cpg, width), (groups, 1))
    w2_bd = bd_mask * w2_tiled
    w3_t = jnp.transpose(W_s3)
    b2r = b_s2.reshape(1, width)
    b3r = b_s3.reshape(1, c_out)
    bar = b_a.reshape(c_out, 1)

    gather_fn = _make_sc_gather(n, width, n * K)

    outs = []
    for bi in range(b):
        idx_b, y_b, z_b = pl.pallas_call(
            _knn_body,
            grid=(n // TILE_A,),
            in_specs=[
                pl.BlockSpec((1, c, TILE_A), lambda ti, bi=bi: (bi, 0, ti)),
                pl.BlockSpec((1, c, n), lambda ti, bi=bi: (bi, 0, 0)),
                pl.BlockSpec((c, width), lambda ti: (0, 0)),
                pl.BlockSpec((c, width), lambda ti: (0, 0)),
                pl.BlockSpec((1, width), lambda ti: (0, 0)),
            ],
            out_specs=[
                pl.BlockSpec((1, K, TILE_A), lambda ti: (0, 0, ti)),
                pl.BlockSpec((1, TILE_A, width), lambda ti: (0, ti, 0)),
                pl.BlockSpec((1, TILE_A, width), lambda ti: (0, ti, 0)),
            ],
            out_shape=[
                jax.ShapeDtypeStruct((1, K, n), jnp.int32),
                jax.ShapeDtypeStruct((1, n, width), jnp.float32),
                jax.ShapeDtypeStruct((1, n, width), jnp.float32),
            ],
        )(x, x, wn_t, wc_t, b1r)

        g_b = gather_fn(y_b.reshape(n, width), idx_b.reshape(n * K))

        out_b = pl.pallas_call(
            _mlp_body,
            grid=(n // TILE_C,),
            in_specs=[
                pl.BlockSpec((K, TILE_C, width), lambda ti: (0, ti, 0)),
                pl.BlockSpec((TILE_C, width), lambda ti: (ti, 0)),
                pl.BlockSpec((1, c, TILE_C), lambda ti, bi=bi: (bi, 0, ti)),
                pl.BlockSpec((width, width), lambda ti: (0, 0)),
                pl.BlockSpec((1, width), lambda ti: (0, 0)),
                pl.BlockSpec((width, c_out), lambda ti: (0, 0)),
                pl.BlockSpec((1, c_out), lambda ti: (0, 0)),
                pl.BlockSpec((c_out, c), lambda ti: (0, 0)),
                pl.BlockSpec((c_out, 1), lambda ti: (0, 0)),
            ],
            out_specs=pl.BlockSpec((c_out, TILE_C), lambda ti: (0, ti)),
            out_shape=jax.ShapeDtypeStruct((c_out, n), jnp.float32),
        )(g_b.reshape(K, n, width), z_b.reshape(n, width), x, w2_bd, b2r,
          w3_t, b3r, W_a, bar)
        outs.append(out_b)

    return jnp.stack(outs)


# speedup vs baseline: 1.0391x; 1.0391x over previous
import functools

import jax
import jax.numpy as jnp
from jax import lax
from jax.experimental import pallas as pl
from jax.experimental.pallas import tpu as pltpu
from jax.experimental.pallas import tpu_sc as plsc

K = 8
TILE_A = 512
TILE_C = 1024


def _knn_body(xc_ref, x_ref, wn_ref, idx_ref, y_ref):
    n_total = x_ref.shape[2]
    xc = xc_ref[0]
    xb = x_ref[0]
    cdims = (((0,), (0,)), ((), ()))
    s = lax.dot_general(xc, xb, cdims,
                        preferred_element_type=jnp.float32)
    inner = -2.0 * s
    xx_all = jnp.sum(xb * xb, axis=0, keepdims=True)
    xx_row = jnp.transpose(
        jnp.sum(xc * xc, axis=0, keepdims=True))
    p = (0.0 - xx_row) - inner - xx_all

    colidx = lax.broadcasted_iota(
        jnp.int32, p.shape, 1).astype(jnp.float32)
    neg = jnp.float32(-jnp.inf)
    big = jnp.float32(n_total)
    picks = []
    for ki in range(K):
        m = jnp.max(p, axis=1, keepdims=True)
        cand = jnp.where(p >= m, colidx, big)
        am = jnp.min(cand, axis=1, keepdims=True)
        picks.append(am)
        if ki + 1 < K:
            p = jnp.where(cand == am, neg, p)
    idx_ref[0] = jnp.transpose(
        jnp.concatenate(picks, axis=1)).astype(jnp.int32)

    y_ref[0] = lax.dot_general(xc, wn_ref[...], cdims,
                               preferred_element_type=jnp.float32)


def _mlp_body(g_ref, x_ref, wc_ref, b1_ref, w2_ref, b2_ref, w3_ref, b3_ref,
              wa_ref, ba_ref, out_ref):
    k, t, c = g_ref.shape
    g = g_ref[...]
    xc = x_ref[0]
    z = (lax.dot_general(xc, wc_ref[...], (((0,), (0,)), ((), ())),
                         preferred_element_type=jnp.float32)
         + b1_ref[...])
    h1 = jnp.maximum(g + z[None, :, :], 0.0)
    h1f = h1.reshape(k * t, c)
    h2 = jnp.maximum(
        jnp.dot(h1f, w2_ref[...], preferred_element_type=jnp.float32)
        + b2_ref[...], 0.0)
    h3 = jnp.maximum(
        jnp.dot(h2, w3_ref[...], preferred_element_type=jnp.float32)
        + b3_ref[...], 0.0).reshape(k, t, c)
    m = h3[0]
    for i in range(1, k):
        m = jnp.maximum(m, h3[i])
    pre = x_ref[0] + jnp.transpose(m)
    out_ref[...] = jnp.maximum(
        jnp.dot(wa_ref[...], pre, preferred_element_type=jnp.float32)
        + ba_ref[...], 0.0)


def _make_sc_gather(n_rows, d, n_idx):
    info = plsc.get_sparse_core_info()
    nc, ns = info.num_cores, info.num_subcores
    nw = nc * ns
    per_w = n_idx // nw
    chunk = 256
    n_chunks = per_w // chunk
    mesh = plsc.VectorSubcoreMesh(core_axis_name="c", subcore_axis_name="s")

    @functools.partial(
        pl.kernel, mesh=mesh,
        out_type=jax.ShapeDtypeStruct((n_idx, d), jnp.float32),
        scratch_types=[
            pltpu.VMEM((per_w,), jnp.int32),
            pltpu.VMEM((chunk, d), jnp.float32),
            pltpu.VMEM((chunk, d), jnp.float32),
            pltpu.SemaphoreType.DMA,
            pltpu.SemaphoreType.DMA,
        ],
    )
    def gather_kernel(table_hbm, idx_hbm, out_hbm, idx_v, rows0, rows1, s0, s1):
        wid = lax.axis_index("s") * nc + lax.axis_index("c")
        base = wid * per_w
        pltpu.sync_copy(idx_hbm.at[pl.ds(base, per_w)], idx_v)
        bufs = (rows0, rows1)
        sems = (s0, s1)
        descs = [None] * n_chunks
        descs[0] = pltpu.async_copy(
            table_hbm.at[idx_v.at[pl.ds(0, chunk)]], bufs[0], sems[0])
        for i in range(n_chunks):
            if i + 1 < n_chunks:
                descs[i + 1] = pltpu.async_copy(
                    table_hbm.at[idx_v.at[pl.ds((i + 1) * chunk, chunk)]],
                    bufs[(i + 1) % 2], sems[(i + 1) % 2])
            descs[i].wait()
            pltpu.sync_copy(bufs[i % 2],
                            out_hbm.at[pl.ds(base + i * chunk, chunk)])

    return gather_kernel


def kernel(x, W_s1, b_s1, W_s2, b_s2, W_s3, b_s3, W_a, b_a):
    b, c, n = x.shape
    groups = 32
    cpg = W_s2.shape[1]
    width = W_s1.shape[0]
    c_out = W_s3.shape[0]

    wn_t = jnp.transpose(W_s1[:, :c])
    wc_t = jnp.transpose(W_s1[:, c:])
    b1r = b_s1.reshape(1, width)

    bd_mask = jnp.kron(jnp.eye(groups, dtype=jnp.float32),
                       jnp.ones((cpg, cpg), jnp.float32))
    w2_tiled = jnp.tile(
        W_s2.reshape(groups, cpg, cpg).transpose(2, 0, 1).reshape(
            cpg, width), (groups, 1))
    w2_bd = bd_mask * w2_tiled
    w3_t = jnp.transpose(W_s3)
    b2r = b_s2.reshape(1, width)
    b3r = b_s3.reshape(1, c_out)
    bar = b_a.reshape(c_out, 1)

    gather_fn = _make_sc_gather(n, width, n * K)

    outs = []
    for bi in range(b):
        idx_b, y_b = pl.pallas_call(
            _knn_body,
            grid=(n // TILE_A,),
            in_specs=[
                pl.BlockSpec((1, c, TILE_A), lambda ti, bi=bi: (bi, 0, ti)),
                pl.BlockSpec((1, c, n), lambda ti, bi=bi: (bi, 0, 0)),
                pl.BlockSpec((c, width), lambda ti: (0, 0)),
            ],
            out_specs=[
                pl.BlockSpec((1, K, TILE_A), lambda ti: (0, 0, ti)),
                pl.BlockSpec((1, TILE_A, width), lambda ti: (0, ti, 0)),
            ],
            out_shape=[
                jax.ShapeDtypeStruct((1, K, n), jnp.int32),
                jax.ShapeDtypeStruct((1, n, width), jnp.float32),
            ],
        )(x, x, wn_t)

        g_b = gather_fn(y_b.reshape(n, width), idx_b.reshape(n * K))

        out_b = pl.pallas_call(
            _mlp_body,
            grid=(n // TILE_C,),
            in_specs=[
                pl.BlockSpec((K, TILE_C, width), lambda ti: (0, ti, 0)),
                pl.BlockSpec((1, c, TILE_C), lambda ti, bi=bi: (bi, 0, ti)),
                pl.BlockSpec((c, width), lambda ti: (0, 0)),
                pl.BlockSpec((1, width), lambda ti: (0, 0)),
                pl.BlockSpec((width, width), lambda ti: (0, 0)),
                pl.BlockSpec((1, width), lambda ti: (0, 0)),
                pl.BlockSpec((width, c_out), lambda ti: (0, 0)),
                pl.BlockSpec((1, c_out), lambda ti: (0, 0)),
                pl.BlockSpec((c_out, c), lambda ti: (0, 0)),
                pl.BlockSpec((c_out, 1), lambda ti: (0, 0)),
            ],
            out_specs=pl.BlockSpec((c_out, TILE_C), lambda ti: (0, ti)),
            out_shape=jax.ShapeDtypeStruct((c_out, n), jnp.float32),
        )(g_b.reshape(K, n, width), x, wc_t, b1r, w2_bd, b2r,
          w3_t, b3r, W_a, bar)
        outs.append(out_b)

    return jnp.stack(outs)
